# baseline (speedup 1.0000x reference)
import functools

import jax
import jax.numpy as jnp
from jax import lax
from jax.experimental import pallas as pl
from jax.experimental.pallas import tpu as pltpu
from jax.experimental.pallas import tpu_sc as plsc

R, C = 256, 128
N = R * C
LOG2N = 15
D = 128
GRP = 64
ORD = 7

NW = 32
BPW = N // NW
ICH = 128


def _lane_partner(a, d):
    bit = (lax.broadcasted_iota(jnp.int32, (R, C), 1) & d) != 0
    return jnp.where(bit, jnp.roll(a, d, axis=1), jnp.roll(a, -d, axis=1))


def _bitonic(key, val):
    i = (lax.broadcasted_iota(jnp.int32, (R, C), 0) * C
         + lax.broadcasted_iota(jnp.int32, (R, C), 1))
    c_io = lax.broadcasted_iota(jnp.int32, (R, C), 1)
    for k in range(1, LOG2N + 1):
        for j in range(k - 1, -1, -1):
            d = 1 << j
            if d >= C:
                m = d // C
                Q = R // (2 * m)
                kv = key.reshape(Q, 2, m, C)
                vv = val.reshape(Q, 2, m, C)
                kA, kB = kv[:, 0], kv[:, 1]
                vA, vB = vv[:, 0], vv[:, 1]
                if (1 << k) >= N:
                    asc = jnp.ones((Q, 1, 1), jnp.bool_)
                else:
                    qq = lax.broadcasted_iota(jnp.int32, (Q, 1, 1), 0)
                    asc = (qq & ((1 << k) // (2 * m * C))) == 0
                g = (kA > kB) | ((kA == kB) & (vA > vB))
                cm = asc ^ g
                key = jnp.stack([jnp.where(cm, kA, kB),
                                 jnp.where(cm, kB, kA)], axis=1).reshape(R, C)
                val = jnp.stack([jnp.where(cm, vA, vB),
                                 jnp.where(cm, vB, vA)], axis=1).reshape(R, C)
            else:
                kp = _lane_partner(key, d)
                vp = _lane_partner(val, d)
                bit = (c_io & d) != 0
                asc = (i & (1 << k)) == 0
                g = (key > kp) | ((key == kp) & (val > vp))
                keep = (asc ^ bit) ^ g
                key = jnp.where(keep, key, kp)
                val = jnp.where(keep, val, vp)
    return key, val


def _morton(b, x, y, z):
    code = jnp.zeros_like(x)
    for i in range(ORD):
        code = (code
                | (((x >> i) & 1) << (3 * i))
                | (((y >> i) & 1) << (3 * i + 1))
                | (((z >> i) & 1) << (3 * i + 2)))
    return code | (b << (3 * ORD))


def _spread3(v):
    out = jnp.zeros_like(v)
    for i in range(ORD):
        out = out | (((v >> i) & 1) << (3 * i))
    return out


def _unspread3(c):
    out = jnp.zeros_like(c)
    for i in range(ORD):
        out = out | (((c >> (3 * i)) & 1) << i)
    return out


_YZ_MASK = sum(0b110 << (3 * i) for i in range(ORD))


def _code2_from_code1(c1):
    y = _unspread3(c1 >> 1)
    z = _unspread3(c1 >> 2)
    return ((c1 & ~_YZ_MASK)
            | (_spread3(y + 1) << 1)
            | (_spread3(z + 1) << 2))


def _sort_a_body(coors_ref, ind1_ref, c2c_ref):
    b = coors_ref[0 * R:1 * R, :]
    x = coors_ref[1 * R:2 * R, :]
    y = coors_ref[2 * R:3 * R, :]
    z = coors_ref[3 * R:4 * R, :]
    code1 = _morton(b, x, y, z)
    iota = (lax.broadcasted_iota(jnp.int32, (R, C), 0) * C
            + lax.broadcasted_iota(jnp.int32, (R, C), 1))
    ks, ind1 = _bitonic(code1, iota)
    ind1_ref[...] = ind1
    c2c_ref[...] = _code2_from_code1(ks)


def _sort_b_body(c2c_ref, ind1_ref, ind2_ref, ind12_ref):
    iota = (lax.broadcasted_iota(jnp.int32, (R, C), 0) * C
            + lax.broadcasted_iota(jnp.int32, (R, C), 1))
    packed = (ind1_ref[...] << 15) | iota
    _, sv = _bitonic(c2c_ref[...], packed)
    ind2_ref[...] = sv >> 15
    ind12_ref[...] = sv & (N - 1)


def _sort_a(coors_t, interpret=False):
    return pl.pallas_call(
        _sort_a_body,
        out_shape=(jax.ShapeDtypeStruct((R, C), jnp.int32),
                   jax.ShapeDtypeStruct((R, C), jnp.int32)),
        interpret=interpret,
    )(coors_t)


def _sort_b(c2c, ind1, interpret=False):
    return pl.pallas_call(
        _sort_b_body,
        out_shape=(jax.ShapeDtypeStruct((R, C), jnp.int32),
                   jax.ShapeDtypeStruct((R, C), jnp.int32)),
        interpret=interpret,
    )(c2c, ind1)


RB = 2048


def _proj_body(pt_ref, wpos0_ref, wpos1_ref, q0_ref, q1_ref):
    pt = pt_ref[...]
    dn = (((0,), (0,)), ((), ()))
    q0_ref[...] = lax.dot_general(pt, wpos0_ref[...], dn,
                                  preferred_element_type=jnp.float32)
    q1_ref[...] = lax.dot_general(pt, wpos1_ref[...], dn,
                                  preferred_element_type=jnp.float32)


def _proj(pts_t8, wpos0, wpos1, interpret=False):
    return pl.pallas_call(
        _proj_body,
        grid=(N // RB,),
        in_specs=[
            pl.BlockSpec((8, RB), lambda i: (0, i)),
            pl.BlockSpec((8, D), lambda i: (0, 0)),
            pl.BlockSpec((8, D), lambda i: (0, 0)),
        ],
        out_specs=(pl.BlockSpec((RB, D), lambda i: (i, 0)),
                   pl.BlockSpec((RB, D), lambda i: (i, 0))),
        out_shape=(jax.ShapeDtypeStruct((N, D), jnp.float32),
                   jax.ShapeDtypeStruct((N, D), jnp.float32)),
        interpret=interpret,
    )(pts_t8, wpos0, wpos1)


def _mlp_body(x_ref, q_ref, w1_ref, w2_ref, b1_ref, b2_ref, o_ref):
    x = x_ref[...]
    qg = q_ref[...].reshape(RB // GRP, GRP, D)
    e = (qg - jnp.mean(qg, axis=1, keepdims=True)).reshape(RB, D)
    h = x * e
    h = jnp.dot(h, w1_ref[...], preferred_element_type=jnp.float32) + b1_ref[...]
    h = jax.nn.gelu(h)
    hg = h.reshape(RB // GRP, GRP, D)
    h = (hg - jnp.mean(hg, axis=1, keepdims=True)).reshape(RB, D)
    h = jnp.dot(h, w2_ref[...], preferred_element_type=jnp.float32) + b2_ref[...]
    o_ref[...] = x + h


def _mlp(x, q, w1, w2, b1, b2, interpret=False):
    grid = (N // RB,)
    return pl.pallas_call(
        _mlp_body,
        grid=grid,
        in_specs=[
            pl.BlockSpec((RB, D), lambda i: (i, 0)),
            pl.BlockSpec((RB, D), lambda i: (i, 0)),
            pl.BlockSpec((D, D), lambda i: (0, 0)),
            pl.BlockSpec((D, D), lambda i: (0, 0)),
            pl.BlockSpec((1, D), lambda i: (0, 0)),
            pl.BlockSpec((1, D), lambda i: (0, 0)),
        ],
        out_specs=pl.BlockSpec((RB, D), lambda i: (i, 0)),
        out_shape=jax.ShapeDtypeStruct((N, D), jnp.float32),
        interpret=interpret,
    )(x, q, w1, w2, b1, b2)


def _sc_mesh():
    return plsc.VectorSubcoreMesh(core_axis_name="c", subcore_axis_name="s")


def _worker_id():
    return lax.axis_index("s") * 2 + lax.axis_index("c")


NT = BPW // ICH


def _wide_body(table, ind, out, idx_v, rows_v, gsem0, gsem1, ssem0, ssem1):
    wid = _worker_id()
    base = wid * BPW
    pltpu.sync_copy(ind.at[pl.ds(wid * (BPW // C), BPW // C)], idx_v)
    gsem = (gsem0, gsem1)
    ssem = (ssem0, ssem1)
    gathers = [None, None]
    stores = [None, None]
    gathers[0] = pltpu.async_copy(table.at[idx_v.at[0]], rows_v.at[0], gsem[0])
    for t in range(NT):
        nxt = (t + 1) % 2
        if t + 1 < NT:
            if stores[nxt] is not None:
                stores[nxt].wait()
                stores[nxt] = None
            gathers[nxt] = pltpu.async_copy(
                table.at[idx_v.at[t + 1]], rows_v.at[nxt], gsem[nxt])
        gathers[t % 2].wait()
        stores[t % 2] = pltpu.async_copy(
            rows_v.at[t % 2], out.at[pl.ds(base + t * ICH, ICH)], ssem[t % 2])
    stores[(NT - 1) % 2].wait()
    if stores[NT % 2] is not None:
        stores[NT % 2].wait()


def _sc_wide(table, ind_2d):
    kern = functools.partial(
        pl.kernel,
        out_type=jax.ShapeDtypeStruct((N, D), jnp.float32),
        mesh=_sc_mesh(),
        scratch_types=[
            pltpu.VMEM((BPW // C, C), jnp.int32),
            pltpu.VMEM((2, ICH, D), jnp.float32),
            pltpu.SemaphoreType.DMA,
            pltpu.SemaphoreType.DMA,
            pltpu.SemaphoreType.DMA,
            pltpu.SemaphoreType.DMA,
        ],
    )
    return kern(_wide_body)(table, ind_2d)


def kernel(vox_feats, pts_coors, Wpos, W1, W2, b1, b2, vox_coors, vox_numbs):
    del vox_numbs
    coors_t = vox_coors.T.reshape(4 * R, C)
    pts_t8 = jnp.pad(pts_coors.T, ((0, 5), (0, 0)))

    wpos_p = jnp.pad(Wpos, ((0, 0), (0, 5), (0, 0)))
    b1r = b1.reshape(2, 1, D)
    b2r = b2.reshape(2, 1, D)

    q0, q1 = _proj(pts_t8, wpos_p[0], wpos_p[1])
    ind1, c2c = _sort_a(coors_t)
    e1 = _sc_wide(q0, ind1)
    x1 = _sc_wide(vox_feats, ind1)
    ind2, ind12 = _sort_b(c2c, ind1)
    e2 = _sc_wide(q1, ind2)
    y = _mlp(x1, e1, W1[0], W2[0], b1r[0], b2r[0])
    x2 = _sc_wide(y, ind12)
    return _mlp(x2, e2, W1[1], W2[1], b1r[1], b2r[1])


# speedup vs baseline: 1.0815x; 1.0815x over previous
import functools

import jax
import jax.numpy as jnp
from jax import lax
from jax.experimental import pallas as pl
from jax.experimental.pallas import tpu as pltpu
from jax.experimental.pallas import tpu_sc as plsc

R, C = 256, 128
N = R * C
LOG2N = 15
D = 128
GRP = 64
ORD = 7

NW = 32
BPW = N // NW
ICH = 128


def _lane_partner(a, d):
    bit = (lax.broadcasted_iota(jnp.int32, (R, C), 1) & d) != 0
    return jnp.where(bit, jnp.roll(a, d, axis=1), jnp.roll(a, -d, axis=1))


def _bitonic(key, val):
    i = (lax.broadcasted_iota(jnp.int32, (R, C), 0) * C
         + lax.broadcasted_iota(jnp.int32, (R, C), 1))
    for k in range(1, LOG2N + 1):
        asc = (i & (1 << k)) == 0
        for j in range(k - 1, -1, -1):
            d = 1 << j
            if d >= C:
                m = d // C
                bit = (lax.broadcasted_iota(jnp.int32, (R, C), 0) & m) != 0
                kp = jnp.where(bit, jnp.roll(key, m, axis=0),
                               jnp.roll(key, -m, axis=0))
                vp = jnp.where(bit, jnp.roll(val, m, axis=0),
                               jnp.roll(val, -m, axis=0))
            else:
                bit = (lax.broadcasted_iota(jnp.int32, (R, C), 1) & d) != 0
                kp = jnp.where(bit, jnp.roll(key, d, axis=1),
                               jnp.roll(key, -d, axis=1))
                vp = jnp.where(bit, jnp.roll(val, d, axis=1),
                               jnp.roll(val, -d, axis=1))
            g = (key > kp) | ((key == kp) & (val > vp))
            keep = (asc ^ bit) ^ g
            key = jnp.where(keep, key, kp)
            val = jnp.where(keep, val, vp)
    return key, val


def _morton(b, x, y, z):
    code = jnp.zeros_like(x)
    for i in range(ORD):
        code = (code
                | (((x >> i) & 1) << (3 * i))
                | (((y >> i) & 1) << (3 * i + 1))
                | (((z >> i) & 1) << (3 * i + 2)))
    return code | (b << (3 * ORD))


def _spread3(v):
    out = jnp.zeros_like(v)
    for i in range(ORD):
        out = out | (((v >> i) & 1) << (3 * i))
    return out


def _unspread3(c):
    out = jnp.zeros_like(c)
    for i in range(ORD):
        out = out | (((c >> (3 * i)) & 1) << i)
    return out


_YZ_MASK = sum(0b110 << (3 * i) for i in range(ORD))


def _code2_from_code1(c1):
    y = _unspread3(c1 >> 1)
    z = _unspread3(c1 >> 2)
    return ((c1 & ~_YZ_MASK)
            | (_spread3(y + 1) << 1)
            | (_spread3(z + 1) << 2))


def _sort_a_body(coors_ref, ind1_ref, c2c_ref):
    b = coors_ref[0 * R:1 * R, :]
    x = coors_ref[1 * R:2 * R, :]
    y = coors_ref[2 * R:3 * R, :]
    z = coors_ref[3 * R:4 * R, :]
    code1 = _morton(b, x, y, z)
    iota = (lax.broadcasted_iota(jnp.int32, (R, C), 0) * C
            + lax.broadcasted_iota(jnp.int32, (R, C), 1))
    ks, ind1 = _bitonic(code1, iota)
    ind1_ref[...] = ind1
    c2c_ref[...] = _code2_from_code1(ks)


def _sort_b_body(c2c_ref, ind1_ref, ind2_ref, ind12_ref):
    iota = (lax.broadcasted_iota(jnp.int32, (R, C), 0) * C
            + lax.broadcasted_iota(jnp.int32, (R, C), 1))
    packed = (ind1_ref[...] << 15) | iota
    _, sv = _bitonic(c2c_ref[...], packed)
    ind2_ref[...] = sv >> 15
    ind12_ref[...] = sv & (N - 1)


def _sort_a(coors_t, interpret=False):
    return pl.pallas_call(
        _sort_a_body,
        out_shape=(jax.ShapeDtypeStruct((R, C), jnp.int32),
                   jax.ShapeDtypeStruct((R, C), jnp.int32)),
        interpret=interpret,
    )(coors_t)


def _sort_b(c2c, ind1, interpret=False):
    return pl.pallas_call(
        _sort_b_body,
        out_shape=(jax.ShapeDtypeStruct((R, C), jnp.int32),
                   jax.ShapeDtypeStruct((R, C), jnp.int32)),
        interpret=interpret,
    )(c2c, ind1)


RB = 2048


def _proj_body(pt_ref, wpos0_ref, wpos1_ref, q0_ref, q1_ref):
    pt = pt_ref[...]
    dn = (((0,), (0,)), ((), ()))
    q0_ref[...] = lax.dot_general(pt, wpos0_ref[...], dn,
                                  preferred_element_type=jnp.float32)
    q1_ref[...] = lax.dot_general(pt, wpos1_ref[...], dn,
                                  preferred_element_type=jnp.float32)


def _proj(pts_t8, wpos0, wpos1, interpret=False):
    return pl.pallas_call(
        _proj_body,
        grid=(N // RB,),
        in_specs=[
            pl.BlockSpec((8, RB), lambda i: (0, i)),
            pl.BlockSpec((8, D), lambda i: (0, 0)),
            pl.BlockSpec((8, D), lambda i: (0, 0)),
        ],
        out_specs=(pl.BlockSpec((RB, D), lambda i: (i, 0)),
                   pl.BlockSpec((RB, D), lambda i: (i, 0))),
        out_shape=(jax.ShapeDtypeStruct((N, D), jnp.float32),
                   jax.ShapeDtypeStruct((N, D), jnp.float32)),
        interpret=interpret,
    )(pts_t8, wpos0, wpos1)


def _mlp_body(x_ref, q_ref, w1_ref, w2_ref, b1_ref, b2_ref, o_ref):
    x = x_ref[...]
    qg = q_ref[...].reshape(RB // GRP, GRP, D)
    e = (qg - jnp.mean(qg, axis=1, keepdims=True)).reshape(RB, D)
    h = x * e
    h = jnp.dot(h, w1_ref[...], preferred_element_type=jnp.float32) + b1_ref[...]
    h = jax.nn.gelu(h)
    hg = h.reshape(RB // GRP, GRP, D)
    h = (hg - jnp.mean(hg, axis=1, keepdims=True)).reshape(RB, D)
    h = jnp.dot(h, w2_ref[...], preferred_element_type=jnp.float32) + b2_ref[...]
    o_ref[...] = x + h


def _mlp(x, q, w1, w2, b1, b2, interpret=False):
    grid = (N // RB,)
    return pl.pallas_call(
        _mlp_body,
        grid=grid,
        in_specs=[
            pl.BlockSpec((RB, D), lambda i: (i, 0)),
            pl.BlockSpec((RB, D), lambda i: (i, 0)),
            pl.BlockSpec((D, D), lambda i: (0, 0)),
            pl.BlockSpec((D, D), lambda i: (0, 0)),
            pl.BlockSpec((1, D), lambda i: (0, 0)),
            pl.BlockSpec((1, D), lambda i: (0, 0)),
        ],
        out_specs=pl.BlockSpec((RB, D), lambda i: (i, 0)),
        out_shape=jax.ShapeDtypeStruct((N, D), jnp.float32),
        interpret=interpret,
    )(x, q, w1, w2, b1, b2)


def _sc_mesh():
    return plsc.VectorSubcoreMesh(core_axis_name="c", subcore_axis_name="s")


def _worker_id():
    return lax.axis_index("s") * 2 + lax.axis_index("c")


NT = BPW // ICH


def _wide_body(table, ind, out, idx_v, rows_v, gsem0, gsem1, ssem0, ssem1):
    wid = _worker_id()
    base = wid * BPW
    pltpu.sync_copy(ind.at[pl.ds(wid * (BPW // C), BPW // C)], idx_v)
    gsem = (gsem0, gsem1)
    ssem = (ssem0, ssem1)
    gathers = [None, None]
    stores = [None, None]
    gathers[0] = pltpu.async_copy(table.at[idx_v.at[0]], rows_v.at[0], gsem[0])
    for t in range(NT):
        nxt = (t + 1) % 2
        if t + 1 < NT:
            if stores[nxt] is not None:
                stores[nxt].wait()
                stores[nxt] = None
            gathers[nxt] = pltpu.async_copy(
                table.at[idx_v.at[t + 1]], rows_v.at[nxt], gsem[nxt])
        gathers[t % 2].wait()
        stores[t % 2] = pltpu.async_copy(
            rows_v.at[t % 2], out.at[pl.ds(base + t * ICH, ICH)], ssem[t % 2])
    stores[(NT - 1) % 2].wait()
    if stores[NT % 2] is not None:
        stores[NT % 2].wait()


def _sc_wide(table, ind_2d):
    kern = functools.partial(
        pl.kernel,
        out_type=jax.ShapeDtypeStruct((N, D), jnp.float32),
        mesh=_sc_mesh(),
        scratch_types=[
            pltpu.VMEM((BPW // C, C), jnp.int32),
            pltpu.VMEM((2, ICH, D), jnp.float32),
            pltpu.SemaphoreType.DMA,
            pltpu.SemaphoreType.DMA,
            pltpu.SemaphoreType.DMA,
            pltpu.SemaphoreType.DMA,
        ],
    )
    return kern(_wide_body)(table, ind_2d)


def kernel(vox_feats, pts_coors, Wpos, W1, W2, b1, b2, vox_coors, vox_numbs):
    del vox_numbs
    coors_t = vox_coors.T.reshape(4 * R, C)
    pts_t8 = jnp.pad(pts_coors.T, ((0, 5), (0, 0)))

    wpos_p = jnp.pad(Wpos, ((0, 0), (0, 5), (0, 0)))
    b1r = b1.reshape(2, 1, D)
    b2r = b2.reshape(2, 1, D)

    q0, q1 = _proj(pts_t8, wpos_p[0], wpos_p[1])
    ind1, c2c = _sort_a(coors_t)
    e1 = _sc_wide(q0, ind1)
    x1 = _sc_wide(vox_feats, ind1)
    ind2, ind12 = _sort_b(c2c, ind1)
    e2 = _sc_wide(q1, ind2)
    y = _mlp(x1, e1, W1[0], W2[0], b1r[0], b2r[0])
    x2 = _sc_wide(y, ind12)
    return _mlp(x2, e2, W1[1], W2[1], b1r[1], b2r[1])


# speedup vs baseline: 1.1576x; 1.0703x over previous
import functools

import jax
import jax.numpy as jnp
from jax import lax
from jax.experimental import pallas as pl
from jax.experimental.pallas import tpu as pltpu
from jax.experimental.pallas import tpu_sc as plsc

R, C = 256, 128
N = R * C
LOG2N = 15
D = 128
GRP = 64
ORD = 7

NW = 32
BPW = N // NW
ICH = 128


def _lane_partner(a, d):
    bit = (lax.broadcasted_iota(jnp.int32, (R, C), 1) & d) != 0
    return jnp.where(bit, jnp.roll(a, d, axis=1), jnp.roll(a, -d, axis=1))


def _bitonic(key, val):
    i = (lax.broadcasted_iota(jnp.int32, (R, C), 0) * C
         + lax.broadcasted_iota(jnp.int32, (R, C), 1))
    for k in range(1, LOG2N + 1):
        asc = (i & (1 << k)) == 0
        for j in range(k - 1, -1, -1):
            d = 1 << j
            if d >= C:
                m = d // C
                bit = (lax.broadcasted_iota(jnp.int32, (R, C), 0) & m) != 0
                kp = jnp.where(bit, jnp.roll(key, m, axis=0),
                               jnp.roll(key, -m, axis=0))
                vp = jnp.where(bit, jnp.roll(val, m, axis=0),
                               jnp.roll(val, -m, axis=0))
            else:
                bit = (lax.broadcasted_iota(jnp.int32, (R, C), 1) & d) != 0
                kp = jnp.where(bit, jnp.roll(key, d, axis=1),
                               jnp.roll(key, -d, axis=1))
                vp = jnp.where(bit, jnp.roll(val, d, axis=1),
                               jnp.roll(val, -d, axis=1))
            g = (key > kp) | ((key == kp) & (val > vp))
            keep = (asc ^ bit) ^ g
            key = jnp.where(keep, key, kp)
            val = jnp.where(keep, val, vp)
    return key, val


def _morton(b, x, y, z):
    code = jnp.zeros_like(x)
    for i in range(ORD):
        code = (code
                | (((x >> i) & 1) << (3 * i))
                | (((y >> i) & 1) << (3 * i + 1))
                | (((z >> i) & 1) << (3 * i + 2)))
    return code | (b << (3 * ORD))


def _spread3(v):
    out = jnp.zeros_like(v)
    for i in range(ORD):
        out = out | (((v >> i) & 1) << (3 * i))
    return out


def _unspread3(c):
    out = jnp.zeros_like(c)
    for i in range(ORD):
        out = out | (((c >> (3 * i)) & 1) << i)
    return out


_YZ_MASK = sum(0b110 << (3 * i) for i in range(ORD))


def _code2_from_code1(c1):
    y = _unspread3(c1 >> 1)
    z = _unspread3(c1 >> 2)
    return ((c1 & ~_YZ_MASK)
            | (_spread3(y + 1) << 1)
            | (_spread3(z + 1) << 2))


def _sort_a_body(coors_ref, ind1_ref, c2c_ref):
    b = coors_ref[0 * R:1 * R, :]
    x = coors_ref[1 * R:2 * R, :]
    y = coors_ref[2 * R:3 * R, :]
    z = coors_ref[3 * R:4 * R, :]
    code1 = _morton(b, x, y, z)
    iota = (lax.broadcasted_iota(jnp.int32, (R, C), 0) * C
            + lax.broadcasted_iota(jnp.int32, (R, C), 1))
    ks, ind1 = _bitonic(code1, iota)
    ind1_ref[...] = ind1
    c2c_ref[...] = _code2_from_code1(ks)


def _sort_b_body(c2c_ref, ind1_ref, ind2_ref, ind12_ref):
    iota = (lax.broadcasted_iota(jnp.int32, (R, C), 0) * C
            + lax.broadcasted_iota(jnp.int32, (R, C), 1))
    packed = (ind1_ref[...] << 15) | iota
    _, sv = _bitonic(c2c_ref[...], packed)
    ind2_ref[...] = sv >> 15
    ind12_ref[...] = sv & (N - 1)


def _sort_a(coors_t, interpret=False):
    return pl.pallas_call(
        _sort_a_body,
        out_shape=(jax.ShapeDtypeStruct((R, C), jnp.int32),
                   jax.ShapeDtypeStruct((R, C), jnp.int32)),
        interpret=interpret,
    )(coors_t)


def _sort_b(c2c, ind1, interpret=False):
    return pl.pallas_call(
        _sort_b_body,
        out_shape=(jax.ShapeDtypeStruct((R, C), jnp.int32),
                   jax.ShapeDtypeStruct((R, C), jnp.int32)),
        interpret=interpret,
    )(c2c, ind1)


RB = 4096


def _proj_body(pt_ref, wpos0_ref, wpos1_ref, q0_ref, q1_ref):
    pt = pt_ref[...]
    dn = (((0,), (0,)), ((), ()))
    q0_ref[...] = lax.dot_general(pt, wpos0_ref[...], dn,
                                  preferred_element_type=jnp.float32)
    q1_ref[...] = lax.dot_general(pt, wpos1_ref[...], dn,
                                  preferred_element_type=jnp.float32)


def _proj(pts_t8, wpos0, wpos1, interpret=False):
    return pl.pallas_call(
        _proj_body,
        grid=(N // RB,),
        in_specs=[
            pl.BlockSpec((8, RB), lambda i: (0, i)),
            pl.BlockSpec((8, D), lambda i: (0, 0)),
            pl.BlockSpec((8, D), lambda i: (0, 0)),
        ],
        out_specs=(pl.BlockSpec((RB, D), lambda i: (i, 0)),
                   pl.BlockSpec((RB, D), lambda i: (i, 0))),
        out_shape=(jax.ShapeDtypeStruct((N, D), jnp.float32),
                   jax.ShapeDtypeStruct((N, D), jnp.float32)),
        interpret=interpret,
    )(pts_t8, wpos0, wpos1)


def _mlp_body(x_ref, q_ref, w1_ref, w2_ref, b1_ref, b2_ref, o_ref):
    x = x_ref[...]
    qg = q_ref[...].reshape(RB // GRP, GRP, D)
    e = (qg - jnp.mean(qg, axis=1, keepdims=True)).reshape(RB, D)
    h = x * e
    h = jnp.dot(h, w1_ref[...], preferred_element_type=jnp.float32) + b1_ref[...]
    h = jax.nn.gelu(h)
    hg = h.reshape(RB // GRP, GRP, D)
    h = (hg - jnp.mean(hg, axis=1, keepdims=True)).reshape(RB, D)
    h = jnp.dot(h, w2_ref[...], preferred_element_type=jnp.float32) + b2_ref[...]
    o_ref[...] = x + h


def _mlp(x, q, w1, w2, b1, b2, interpret=False):
    grid = (N // RB,)
    return pl.pallas_call(
        _mlp_body,
        grid=grid,
        in_specs=[
            pl.BlockSpec((RB, D), lambda i: (i, 0)),
            pl.BlockSpec((RB, D), lambda i: (i, 0)),
            pl.BlockSpec((D, D), lambda i: (0, 0)),
            pl.BlockSpec((D, D), lambda i: (0, 0)),
            pl.BlockSpec((1, D), lambda i: (0, 0)),
            pl.BlockSpec((1, D), lambda i: (0, 0)),
        ],
        out_specs=pl.BlockSpec((RB, D), lambda i: (i, 0)),
        out_shape=jax.ShapeDtypeStruct((N, D), jnp.float32),
        interpret=interpret,
    )(x, q, w1, w2, b1, b2)


def _sc_mesh():
    return plsc.VectorSubcoreMesh(core_axis_name="c", subcore_axis_name="s")


def _worker_id():
    return lax.axis_index("s") * 2 + lax.axis_index("c")


NT = BPW // ICH


def _wide_body(table, ind, out, idx_v, rows_v, gsem0, gsem1, ssem0, ssem1):
    wid = _worker_id()
    base = wid * BPW
    pltpu.sync_copy(ind.at[pl.ds(wid * (BPW // C), BPW // C)], idx_v)
    gsem = (gsem0, gsem1)
    ssem = (ssem0, ssem1)
    gathers = [None, None]
    stores = [None, None]
    gathers[0] = pltpu.async_copy(table.at[idx_v.at[0]], rows_v.at[0], gsem[0])
    for t in range(NT):
        nxt = (t + 1) % 2
        if t + 1 < NT:
            if stores[nxt] is not None:
                stores[nxt].wait()
                stores[nxt] = None
            gathers[nxt] = pltpu.async_copy(
                table.at[idx_v.at[t + 1]], rows_v.at[nxt], gsem[nxt])
        gathers[t % 2].wait()
        stores[t % 2] = pltpu.async_copy(
            rows_v.at[t % 2], out.at[pl.ds(base + t * ICH, ICH)], ssem[t % 2])
    stores[(NT - 1) % 2].wait()
    if stores[NT % 2] is not None:
        stores[NT % 2].wait()


def _sc_wide(table, ind_2d):
    kern = functools.partial(
        pl.kernel,
        out_type=jax.ShapeDtypeStruct((N, D), table.dtype),
        mesh=_sc_mesh(),
        scratch_types=[
            pltpu.VMEM((BPW // C, C), jnp.int32),
            pltpu.VMEM((2, ICH, D), table.dtype),
            pltpu.SemaphoreType.DMA,
            pltpu.SemaphoreType.DMA,
            pltpu.SemaphoreType.DMA,
            pltpu.SemaphoreType.DMA,
        ],
    )
    return kern(_wide_body)(table, ind_2d)


def kernel(vox_feats, pts_coors, Wpos, W1, W2, b1, b2, vox_coors, vox_numbs):
    del vox_numbs
    coors_t = vox_coors.T.reshape(4 * R, C)
    pts_t8 = jnp.pad(pts_coors.T, ((0, 5), (0, 0)))

    wpos_p = jnp.pad(Wpos, ((0, 0), (0, 5), (0, 0)))
    b1r = b1.reshape(2, 1, D)
    b2r = b2.reshape(2, 1, D)

    q0, q1 = _proj(pts_t8, wpos_p[0], wpos_p[1])
    ind1, c2c = _sort_a(coors_t)
    e1 = _sc_wide(q0, ind1)
    x1 = _sc_wide(vox_feats, ind1)
    ind2, ind12 = _sort_b(c2c, ind1)
    e2 = _sc_wide(q1, ind2)
    y = _mlp(x1, e1, W1[0], W2[0], b1r[0], b2r[0])
    x2 = _sc_wide(y, ind12)
    return _mlp(x2, e2, W1[1], W2[1], b1r[1], b2r[1])


# speedup vs baseline: 1.1627x; 1.0044x over previous
import functools

import jax
import jax.numpy as jnp
from jax import lax
from jax.experimental import pallas as pl
from jax.experimental.pallas import tpu as pltpu
from jax.experimental.pallas import tpu_sc as plsc

R, C = 256, 128
N = R * C
LOG2N = 15
D = 128
GRP = 64
ORD = 7

NW = 32
BPW = N // NW
ICH = 128


def _lane_partner(a, d):
    bit = (lax.broadcasted_iota(jnp.int32, (R, C), 1) & d) != 0
    return jnp.where(bit, jnp.roll(a, d, axis=1), jnp.roll(a, -d, axis=1))


def _bitonic(key, val):
    i = (lax.broadcasted_iota(jnp.int32, (R, C), 0) * C
         + lax.broadcasted_iota(jnp.int32, (R, C), 1))
    for k in range(1, LOG2N + 1):
        asc = (i & (1 << k)) == 0
        for j in range(k - 1, -1, -1):
            d = 1 << j
            if d >= C:
                m = d // C
                bit = (lax.broadcasted_iota(jnp.int32, (R, C), 0) & m) != 0
                kp = jnp.where(bit, jnp.roll(key, m, axis=0),
                               jnp.roll(key, -m, axis=0))
                vp = jnp.where(bit, jnp.roll(val, m, axis=0),
                               jnp.roll(val, -m, axis=0))
            else:
                bit = (lax.broadcasted_iota(jnp.int32, (R, C), 1) & d) != 0
                kp = jnp.where(bit, jnp.roll(key, d, axis=1),
                               jnp.roll(key, -d, axis=1))
                vp = jnp.where(bit, jnp.roll(val, d, axis=1),
                               jnp.roll(val, -d, axis=1))
            g = (key > kp) | ((key == kp) & (val > vp))
            keep = (asc ^ bit) ^ g
            key = jnp.where(keep, key, kp)
            val = jnp.where(keep, val, vp)
    return key, val


def _morton(b, x, y, z):
    code = jnp.zeros_like(x)
    for i in range(ORD):
        code = (code
                | (((x >> i) & 1) << (3 * i))
                | (((y >> i) & 1) << (3 * i + 1))
                | (((z >> i) & 1) << (3 * i + 2)))
    return code | (b << (3 * ORD))


def _spread3(v):
    out = jnp.zeros_like(v)
    for i in range(ORD):
        out = out | (((v >> i) & 1) << (3 * i))
    return out


def _unspread3(c):
    out = jnp.zeros_like(c)
    for i in range(ORD):
        out = out | (((c >> (3 * i)) & 1) << i)
    return out


_YZ_MASK = sum(0b110 << (3 * i) for i in range(ORD))


def _code2_from_code1(c1):
    y = _unspread3(c1 >> 1)
    z = _unspread3(c1 >> 2)
    return ((c1 & ~_YZ_MASK)
            | (_spread3(y + 1) << 1)
            | (_spread3(z + 1) << 2))


def _sort_a_body(coors_ref, ind1_ref, c2c_ref):
    b = coors_ref[0 * R:1 * R, :]
    x = coors_ref[1 * R:2 * R, :]
    y = coors_ref[2 * R:3 * R, :]
    z = coors_ref[3 * R:4 * R, :]
    code1 = _morton(b, x, y, z)
    iota = (lax.broadcasted_iota(jnp.int32, (R, C), 0) * C
            + lax.broadcasted_iota(jnp.int32, (R, C), 1))
    ks, ind1 = _bitonic(code1, iota)
    ind1_ref[...] = ind1
    c2c_ref[...] = _code2_from_code1(ks)


def _sort_b_body(c2c_ref, ind1_ref, ind2_ref, ind12_ref):
    iota = (lax.broadcasted_iota(jnp.int32, (R, C), 0) * C
            + lax.broadcasted_iota(jnp.int32, (R, C), 1))
    packed = (ind1_ref[...] << 15) | iota
    _, sv = _bitonic(c2c_ref[...], packed)
    ind2_ref[...] = sv >> 15
    ind12_ref[...] = sv & (N - 1)


def _sort_a(coors_t, interpret=False):
    return pl.pallas_call(
        _sort_a_body,
        out_shape=(jax.ShapeDtypeStruct((R, C), jnp.int32),
                   jax.ShapeDtypeStruct((R, C), jnp.int32)),
        interpret=interpret,
    )(coors_t)


def _sort_b(c2c, ind1, interpret=False):
    return pl.pallas_call(
        _sort_b_body,
        out_shape=(jax.ShapeDtypeStruct((R, C), jnp.int32),
                   jax.ShapeDtypeStruct((R, C), jnp.int32)),
        interpret=interpret,
    )(c2c, ind1)


RB = 8192


def _proj_body(pt_ref, wpos0_ref, wpos1_ref, q0_ref, q1_ref):
    pt = pt_ref[...]
    dn = (((0,), (0,)), ((), ()))
    q0_ref[...] = lax.dot_general(pt, wpos0_ref[...], dn,
                                  preferred_element_type=jnp.float32)
    q1_ref[...] = lax.dot_general(pt, wpos1_ref[...], dn,
                                  preferred_element_type=jnp.float32)


def _proj(pts_t8, wpos0, wpos1, interpret=False):
    return pl.pallas_call(
        _proj_body,
        grid=(N // RB,),
        in_specs=[
            pl.BlockSpec((8, RB), lambda i: (0, i)),
            pl.BlockSpec((8, D), lambda i: (0, 0)),
            pl.BlockSpec((8, D), lambda i: (0, 0)),
        ],
        out_specs=(pl.BlockSpec((RB, D), lambda i: (i, 0)),
                   pl.BlockSpec((RB, D), lambda i: (i, 0))),
        out_shape=(jax.ShapeDtypeStruct((N, D), jnp.float32),
                   jax.ShapeDtypeStruct((N, D), jnp.float32)),
        interpret=interpret,
    )(pts_t8, wpos0, wpos1)


def _mlp_body(x_ref, q_ref, w1_ref, w2_ref, b1_ref, b2_ref, o_ref):
    x = x_ref[...]
    qg = q_ref[...].reshape(RB // GRP, GRP, D)
    e = (qg - jnp.mean(qg, axis=1, keepdims=True)).reshape(RB, D)
    h = x * e
    h = jnp.dot(h, w1_ref[...], preferred_element_type=jnp.float32) + b1_ref[...]
    h = jax.nn.gelu(h)
    hg = h.reshape(RB // GRP, GRP, D)
    h = (hg - jnp.mean(hg, axis=1, keepdims=True)).reshape(RB, D)
    h = jnp.dot(h, w2_ref[...], preferred_element_type=jnp.float32) + b2_ref[...]
    o_ref[...] = x + h


def _mlp(x, q, w1, w2, b1, b2, interpret=False):
    grid = (N // RB,)
    return pl.pallas_call(
        _mlp_body,
        grid=grid,
        in_specs=[
            pl.BlockSpec((RB, D), lambda i: (i, 0)),
            pl.BlockSpec((RB, D), lambda i: (i, 0)),
            pl.BlockSpec((D, D), lambda i: (0, 0)),
            pl.BlockSpec((D, D), lambda i: (0, 0)),
            pl.BlockSpec((1, D), lambda i: (0, 0)),
            pl.BlockSpec((1, D), lambda i: (0, 0)),
        ],
        out_specs=pl.BlockSpec((RB, D), lambda i: (i, 0)),
        out_shape=jax.ShapeDtypeStruct((N, D), jnp.float32),
        interpret=interpret,
    )(x, q, w1, w2, b1, b2)


def _sc_mesh():
    return plsc.VectorSubcoreMesh(core_axis_name="c", subcore_axis_name="s")


def _worker_id():
    return lax.axis_index("s") * 2 + lax.axis_index("c")


NT = BPW // ICH


def _wide_body(table, ind, out, idx_v, rows_v, gsem0, gsem1, ssem0, ssem1):
    wid = _worker_id()
    base = wid * BPW
    pltpu.sync_copy(ind.at[pl.ds(wid * (BPW // C), BPW // C)], idx_v)
    gsem = (gsem0, gsem1)
    ssem = (ssem0, ssem1)
    gathers = [None, None]
    stores = [None, None]
    gathers[0] = pltpu.async_copy(table.at[idx_v.at[0]], rows_v.at[0], gsem[0])
    for t in range(NT):
        nxt = (t + 1) % 2
        if t + 1 < NT:
            if stores[nxt] is not None:
                stores[nxt].wait()
                stores[nxt] = None
            gathers[nxt] = pltpu.async_copy(
                table.at[idx_v.at[t + 1]], rows_v.at[nxt], gsem[nxt])
        gathers[t % 2].wait()
        stores[t % 2] = pltpu.async_copy(
            rows_v.at[t % 2], out.at[pl.ds(base + t * ICH, ICH)], ssem[t % 2])
    stores[(NT - 1) % 2].wait()
    if stores[NT % 2] is not None:
        stores[NT % 2].wait()


def _sc_wide(table, ind_2d):
    kern = functools.partial(
        pl.kernel,
        out_type=jax.ShapeDtypeStruct((N, D), table.dtype),
        mesh=_sc_mesh(),
        scratch_types=[
            pltpu.VMEM((BPW // C, C), jnp.int32),
            pltpu.VMEM((2, ICH, D), table.dtype),
            pltpu.SemaphoreType.DMA,
            pltpu.SemaphoreType.DMA,
            pltpu.SemaphoreType.DMA,
            pltpu.SemaphoreType.DMA,
        ],
    )
    return kern(_wide_body)(table, ind_2d)


def kernel(vox_feats, pts_coors, Wpos, W1, W2, b1, b2, vox_coors, vox_numbs):
    del vox_numbs
    coors_t = vox_coors.T.reshape(4 * R, C)
    pts_t8 = jnp.pad(pts_coors.T, ((0, 5), (0, 0)))

    wpos_p = jnp.pad(Wpos, ((0, 0), (0, 5), (0, 0)))
    b1r = b1.reshape(2, 1, D)
    b2r = b2.reshape(2, 1, D)

    q0, q1 = _proj(pts_t8, wpos_p[0], wpos_p[1])
    ind1, c2c = _sort_a(coors_t)
    e1 = _sc_wide(q0, ind1)
    x1 = _sc_wide(vox_feats, ind1)
    ind2, ind12 = _sort_b(c2c, ind1)
    e2 = _sc_wide(q1, ind2)
    y = _mlp(x1, e1, W1[0], W2[0], b1r[0], b2r[0])
    x2 = _sc_wide(y, ind12)
    return _mlp(x2, e2, W1[1], W2[1], b1r[1], b2r[1])


# speedup vs baseline: 1.1679x; 1.0044x over previous
import functools

import jax
import jax.numpy as jnp
from jax import lax
from jax.experimental import pallas as pl
from jax.experimental.pallas import tpu as pltpu
from jax.experimental.pallas import tpu_sc as plsc

R, C = 256, 128
N = R * C
LOG2N = 15
D = 128
GRP = 64
ORD = 7

NW = 32
BPW = N // NW
ICH = 128


def _lane_partner(a, d):
    bit = (lax.broadcasted_iota(jnp.int32, (R, C), 1) & d) != 0
    return jnp.where(bit, jnp.roll(a, d, axis=1), jnp.roll(a, -d, axis=1))


def _bitonic(key, val):
    i = (lax.broadcasted_iota(jnp.int32, (R, C), 0) * C
         + lax.broadcasted_iota(jnp.int32, (R, C), 1))
    for k in range(1, LOG2N + 1):
        asc = (i & (1 << k)) == 0
        for j in range(k - 1, -1, -1):
            d = 1 << j
            if d >= C:
                m = d // C
                bit = (lax.broadcasted_iota(jnp.int32, (R, C), 0) & m) != 0
                kp = jnp.where(bit, jnp.roll(key, m, axis=0),
                               jnp.roll(key, -m, axis=0))
                vp = jnp.where(bit, jnp.roll(val, m, axis=0),
                               jnp.roll(val, -m, axis=0))
                g = (key > kp) | ((key == kp) & (val > vp))
                keep = (asc ^ bit) ^ g
                key = jnp.where(keep, key, kp)
                val = jnp.where(keep, val, vp)
            else:
                bit = (lax.broadcasted_iota(jnp.int32, (64, C), 1) & d) != 0
                nk, nv = [], []
                for h in range(R // 64):
                    ks = key[h * 64:(h + 1) * 64]
                    vs = val[h * 64:(h + 1) * 64]
                    ah = asc[h * 64:(h + 1) * 64]
                    kp = jnp.where(bit, jnp.roll(ks, d, axis=1),
                                   jnp.roll(ks, -d, axis=1))
                    vp = jnp.where(bit, jnp.roll(vs, d, axis=1),
                                   jnp.roll(vs, -d, axis=1))
                    g = (ks > kp) | ((ks == kp) & (vs > vp))
                    keep = (ah ^ bit) ^ g
                    nk.append(jnp.where(keep, ks, kp))
                    nv.append(jnp.where(keep, vs, vp))
                key = jnp.concatenate(nk, axis=0)
                val = jnp.concatenate(nv, axis=0)
    return key, val


def _morton(b, x, y, z):
    code = jnp.zeros_like(x)
    for i in range(ORD):
        code = (code
                | (((x >> i) & 1) << (3 * i))
                | (((y >> i) & 1) << (3 * i + 1))
                | (((z >> i) & 1) << (3 * i + 2)))
    return code | (b << (3 * ORD))


def _spread3(v):
    out = jnp.zeros_like(v)
    for i in range(ORD):
        out = out | (((v >> i) & 1) << (3 * i))
    return out


def _unspread3(c):
    out = jnp.zeros_like(c)
    for i in range(ORD):
        out = out | (((c >> (3 * i)) & 1) << i)
    return out


_YZ_MASK = sum(0b110 << (3 * i) for i in range(ORD))


def _code2_from_code1(c1):
    y = _unspread3(c1 >> 1)
    z = _unspread3(c1 >> 2)
    return ((c1 & ~_YZ_MASK)
            | (_spread3(y + 1) << 1)
            | (_spread3(z + 1) << 2))


def _sort_a_body(coors_ref, ind1_ref, c2c_ref):
    b = coors_ref[0 * R:1 * R, :]
    x = coors_ref[1 * R:2 * R, :]
    y = coors_ref[2 * R:3 * R, :]
    z = coors_ref[3 * R:4 * R, :]
    code1 = _morton(b, x, y, z)
    iota = (lax.broadcasted_iota(jnp.int32, (R, C), 0) * C
            + lax.broadcasted_iota(jnp.int32, (R, C), 1))
    ks, ind1 = _bitonic(code1, iota)
    ind1_ref[...] = ind1
    c2c_ref[...] = _code2_from_code1(ks)


def _sort_b_body(c2c_ref, ind1_ref, ind2_ref, ind12_ref):
    iota = (lax.broadcasted_iota(jnp.int32, (R, C), 0) * C
            + lax.broadcasted_iota(jnp.int32, (R, C), 1))
    packed = (ind1_ref[...] << 15) | iota
    _, sv = _bitonic(c2c_ref[...], packed)
    ind2_ref[...] = sv >> 15
    ind12_ref[...] = sv & (N - 1)


def _sort_a(coors_t, interpret=False):
    return pl.pallas_call(
        _sort_a_body,
        out_shape=(jax.ShapeDtypeStruct((R, C), jnp.int32),
                   jax.ShapeDtypeStruct((R, C), jnp.int32)),
        interpret=interpret,
    )(coors_t)


def _sort_b(c2c, ind1, interpret=False):
    return pl.pallas_call(
        _sort_b_body,
        out_shape=(jax.ShapeDtypeStruct((R, C), jnp.int32),
                   jax.ShapeDtypeStruct((R, C), jnp.int32)),
        interpret=interpret,
    )(c2c, ind1)


RB = 8192


def _proj_body(pt_ref, wpos0_ref, wpos1_ref, q0_ref, q1_ref):
    pt = pt_ref[...]
    dn = (((0,), (0,)), ((), ()))
    q0_ref[...] = lax.dot_general(pt, wpos0_ref[...], dn,
                                  preferred_element_type=jnp.float32)
    q1_ref[...] = lax.dot_general(pt, wpos1_ref[...], dn,
                                  preferred_element_type=jnp.float32)


def _proj(pts_t8, wpos0, wpos1, interpret=False):
    return pl.pallas_call(
        _proj_body,
        grid=(N // RB,),
        in_specs=[
            pl.BlockSpec((8, RB), lambda i: (0, i)),
            pl.BlockSpec((8, D), lambda i: (0, 0)),
            pl.BlockSpec((8, D), lambda i: (0, 0)),
        ],
        out_specs=(pl.BlockSpec((RB, D), lambda i: (i, 0)),
                   pl.BlockSpec((RB, D), lambda i: (i, 0))),
        out_shape=(jax.ShapeDtypeStruct((N, D), jnp.float32),
                   jax.ShapeDtypeStruct((N, D), jnp.float32)),
        interpret=interpret,
    )(pts_t8, wpos0, wpos1)


def _mlp_body(x_ref, q_ref, w1_ref, w2_ref, b1_ref, b2_ref, o_ref):
    x = x_ref[...]
    qg = q_ref[...].reshape(RB // GRP, GRP, D)
    e = (qg - jnp.mean(qg, axis=1, keepdims=True)).reshape(RB, D)
    h = x * e
    h = jnp.dot(h, w1_ref[...], preferred_element_type=jnp.float32) + b1_ref[...]
    h = jax.nn.gelu(h)
    hg = h.reshape(RB // GRP, GRP, D)
    h = (hg - jnp.mean(hg, axis=1, keepdims=True)).reshape(RB, D)
    h = jnp.dot(h, w2_ref[...], preferred_element_type=jnp.float32) + b2_ref[...]
    o_ref[...] = x + h


def _mlp(x, q, w1, w2, b1, b2, interpret=False):
    grid = (N // RB,)
    return pl.pallas_call(
        _mlp_body,
        grid=grid,
        in_specs=[
            pl.BlockSpec((RB, D), lambda i: (i, 0)),
            pl.BlockSpec((RB, D), lambda i: (i, 0)),
            pl.BlockSpec((D, D), lambda i: (0, 0)),
            pl.BlockSpec((D, D), lambda i: (0, 0)),
            pl.BlockSpec((1, D), lambda i: (0, 0)),
            pl.BlockSpec((1, D), lambda i: (0, 0)),
        ],
        out_specs=pl.BlockSpec((RB, D), lambda i: (i, 0)),
        out_shape=jax.ShapeDtypeStruct((N, D), jnp.float32),
        interpret=interpret,
    )(x, q, w1, w2, b1, b2)


def _sc_mesh():
    return plsc.VectorSubcoreMesh(core_axis_name="c", subcore_axis_name="s")


def _worker_id():
    return lax.axis_index("s") * 2 + lax.axis_index("c")


NT = BPW // ICH


def _wide_body(table, ind, out, idx_v, rows_v, gsem0, gsem1, ssem0, ssem1):
    wid = _worker_id()
    base = wid * BPW
    pltpu.sync_copy(ind.at[pl.ds(wid * (BPW // C), BPW // C)], idx_v)
    gsem = (gsem0, gsem1)
    ssem = (ssem0, ssem1)
    gathers = [None, None]
    stores = [None, None]
    gathers[0] = pltpu.async_copy(table.at[idx_v.at[0]], rows_v.at[0], gsem[0])
    for t in range(NT):
        nxt = (t + 1) % 2
        if t + 1 < NT:
            if stores[nxt] is not None:
                stores[nxt].wait()
                stores[nxt] = None
            gathers[nxt] = pltpu.async_copy(
                table.at[idx_v.at[t + 1]], rows_v.at[nxt], gsem[nxt])
        gathers[t % 2].wait()
        stores[t % 2] = pltpu.async_copy(
            rows_v.at[t % 2], out.at[pl.ds(base + t * ICH, ICH)], ssem[t % 2])
    stores[(NT - 1) % 2].wait()
    if stores[NT % 2] is not None:
        stores[NT % 2].wait()


def _sc_wide(table, ind_2d):
    kern = functools.partial(
        pl.kernel,
        out_type=jax.ShapeDtypeStruct((N, D), table.dtype),
        mesh=_sc_mesh(),
        scratch_types=[
            pltpu.VMEM((BPW // C, C), jnp.int32),
            pltpu.VMEM((2, ICH, D), table.dtype),
            pltpu.SemaphoreType.DMA,
            pltpu.SemaphoreType.DMA,
            pltpu.SemaphoreType.DMA,
            pltpu.SemaphoreType.DMA,
        ],
    )
    return kern(_wide_body)(table, ind_2d)


def kernel(vox_feats, pts_coors, Wpos, W1, W2, b1, b2, vox_coors, vox_numbs):
    del vox_numbs
    coors_t = vox_coors.T.reshape(4 * R, C)
    pts_t8 = jnp.pad(pts_coors.T, ((0, 5), (0, 0)))

    wpos_p = jnp.pad(Wpos, ((0, 0), (0, 5), (0, 0)))
    b1r = b1.reshape(2, 1, D)
    b2r = b2.reshape(2, 1, D)

    q0, q1 = _proj(pts_t8, wpos_p[0], wpos_p[1])
    ind1, c2c = _sort_a(coors_t)
    e1 = _sc_wide(q0, ind1)
    x1 = _sc_wide(vox_feats, ind1)
    ind2, ind12 = _sort_b(c2c, ind1)
    e2 = _sc_wide(q1, ind2)
    y = _mlp(x1, e1, W1[0], W2[0], b1r[0], b2r[0])
    x2 = _sc_wide(y, ind12)
    return _mlp(x2, e2, W1[1], W2[1], b1r[1], b2r[1])
